# Initial kernel scaffold; baseline (speedup 1.0000x reference)
import jax
import jax.numpy as jnp
from jax.experimental import pallas as pl


def kernel(x, edge_index, batch, strategy, Wl0, bl0, Wr0, Wl1, bl1, Wr1, W1, b1, W2, b2, W3, b3):
    raise NotImplementedError("write your pallas kernel here")


# speedup vs baseline: 6.8305x; 6.8305x over previous
import functools

import jax
import jax.numpy as jnp
from jax import lax
from jax.experimental import pallas as pl
from jax.experimental.pallas import tpu as pltpu
from jax.experimental.pallas import tpu_sc as plsc

_N, _E, _D, _B = 10000, 320000, 128, 64
_NC, _NS = 2, 16
_NW = _NC * _NS
_EW = _E // _NW
_C = 80
_K = _EW // _C
_RPS = _N // _NS

_PREC = jax.lax.Precision.HIGHEST


@functools.partial(
    pl.kernel,
    mesh=plsc.VectorSubcoreMesh(core_axis_name="c", subcore_axis_name="s"),
    out_type=jax.ShapeDtypeStruct((_NC, _NS, _RPS, _D), jnp.float32),
    scratch_types=[
        pltpu.VMEM((_K, _C), jnp.int32),
        pltpu.VMEM((_K, _C), jnp.int32),
        pltpu.VMEM((_C, _D), jnp.float32),
        pltpu.VMEM_SHARED((_N, _D), jnp.float32),
        pltpu.SemaphoreType.DMA,
    ],
)
def _edge_agg(h_hbm, src_hbm, dst_hbm, zeros_hbm, out_hbm,
              sidx_v, didx_v, rows_v, acc_sh, sem):
    c = lax.axis_index("c")
    s = lax.axis_index("s")
    wid = s * _NC + c
    pltpu.sync_copy(zeros_hbm, acc_sh.at[pl.ds(s * _RPS, _RPS)])
    pltpu.sync_copy(src_hbm.at[wid], sidx_v)
    pltpu.sync_copy(dst_hbm.at[wid], didx_v)
    plsc.subcore_barrier()

    def body(j, carry):
        pltpu.async_copy(h_hbm.at[sidx_v.at[j]], rows_v, sem).wait()
        pltpu.sync_copy(rows_v, acc_sh.at[didx_v.at[j]], add=True)
        return carry

    lax.fori_loop(0, _K, body, 0)
    plsc.subcore_barrier()
    pltpu.sync_copy(acc_sh.at[pl.ds(s * _RPS, _RPS)], out_hbm.at[c, s])


def _dense_body(a_ref, h_ref, wl_ref, wr_ref, bl_ref, o_ref):
    agg = a_ref[0] + a_ref[1]
    o_ref[...] = jnp.maximum(
        jax.lax.dot(agg, wl_ref[...], precision=_PREC)
        + jax.lax.dot(h_ref[...], wr_ref[...], precision=_PREC)
        + bl_ref[...],
        0.0,
    )


def _dense(a, h, wl, wr, bl):
    bs = 1000
    grid = _N // bs
    return pl.pallas_call(
        _dense_body,
        grid=(grid,),
        in_specs=[
            pl.BlockSpec((_NC, bs, _D), lambda i: (0, i, 0)),
            pl.BlockSpec((bs, _D), lambda i: (i, 0)),
            pl.BlockSpec((_D, _D), lambda i: (0, 0)),
            pl.BlockSpec((_D, _D), lambda i: (0, 0)),
            pl.BlockSpec((1, _D), lambda i: (0, 0)),
        ],
        out_specs=pl.BlockSpec((bs, _D), lambda i: (i, 0)),
        out_shape=jax.ShapeDtypeStruct((_N, _D), jnp.float32),
    )(a, h, wl, wr, bl)


def _pool_head_body(h_ref, b_ref, strat_ref, w1a_ref, w1b_ref, b1_ref,
                    w2_ref, b2_ref, w3_ref, b3_ref, o_ref, g_acc):
    i = pl.program_id(0)

    @pl.when(i == 0)
    def _():
        g_acc[...] = jnp.zeros_like(g_acc)

    bs = h_ref.shape[0]
    batch_row = b_ref[0]
    giota = jax.lax.broadcasted_iota(jnp.int32, (_B, bs), 0)
    mask = jnp.where(giota == batch_row, 1.0, 0.0)
    g_acc[...] += jax.lax.dot(mask, h_ref[...], precision=_PREC)

    @pl.when(i == pl.num_programs(0) - 1)
    def _():
        o1 = (jax.lax.dot(g_acc[...], w1a_ref[...], precision=_PREC)
              + strat_ref[...] * w1b_ref[...] + b1_ref[...])
        o1 = jnp.maximum(o1, 0.0)
        o2 = jnp.maximum(
            jax.lax.dot(o1, w2_ref[...], precision=_PREC) + b2_ref[...], 0.0)
        o_ref[...] = jax.lax.dot(o2, w3_ref[...], precision=_PREC) + b3_ref[...]


def _pool_head(h, batch3, strategy, w1a, w1b, b1, w2, b2, w3, b3):
    bs = 1000
    grid = _N // bs
    return pl.pallas_call(
        _pool_head_body,
        grid=(grid,),
        in_specs=[
            pl.BlockSpec((bs, _D), lambda i: (i, 0)),
            pl.BlockSpec((1, 1, bs), lambda i: (i, 0, 0)),
            pl.BlockSpec((_B, 1), lambda i: (0, 0)),
            pl.BlockSpec((_D, 64), lambda i: (0, 0)),
            pl.BlockSpec((1, 64), lambda i: (0, 0)),
            pl.BlockSpec((1, 64), lambda i: (0, 0)),
            pl.BlockSpec((64, 32), lambda i: (0, 0)),
            pl.BlockSpec((1, 32), lambda i: (0, 0)),
            pl.BlockSpec((32, 6), lambda i: (0, 0)),
            pl.BlockSpec((1, 6), lambda i: (0, 0)),
        ],
        out_specs=pl.BlockSpec((_B, 6), lambda i: (0, 0)),
        out_shape=jax.ShapeDtypeStruct((_B, 6), jnp.float32),
        scratch_shapes=[pltpu.VMEM((_B, _D), jnp.float32)],
    )(h, batch3, strategy, w1a, w1b, b1, w2, b2, w3, b3)


def kernel(x, edge_index, batch, strategy, Wl0, bl0, Wr0, Wl1, bl1, Wr1,
           W1, b1, W2, b2, W3, b3):
    src = edge_index[0].reshape(_NW, _K, _C)
    dst = edge_index[1].reshape(_NW, _K, _C)
    zeros = jnp.zeros((_RPS, _D), jnp.float32)

    a0 = _edge_agg(x, src, dst, zeros).reshape(_NC, _N, _D)
    h1 = _dense(a0, x, Wl0, Wr0, bl0.reshape(1, _D))
    a1 = _edge_agg(h1, src, dst, zeros).reshape(_NC, _N, _D)
    h2 = _dense(a1, h1, Wl1, Wr1, bl1.reshape(1, _D))

    batch3 = batch.reshape(_N // 1000, 1, 1000)
    return _pool_head(h2, batch3, strategy,
                      W1[:_D], W1[_D:], b1.reshape(1, 64),
                      W2, b2.reshape(1, 32), W3, b3.reshape(1, 6))


# speedup vs baseline: 10.8105x; 1.5827x over previous
import functools

import jax
import jax.numpy as jnp
from jax import lax
from jax.experimental import pallas as pl
from jax.experimental.pallas import tpu as pltpu
from jax.experimental.pallas import tpu_sc as plsc

_N, _E, _D, _B = 10000, 320000, 128, 64
_NC, _NS = 2, 16
_NW = _NC * _NS
_EW = _E // _NW
_C = 40
_K = _EW // _C
_RPS = _N // _NS

_PREC = jax.lax.Precision.HIGHEST
_NBUF = 5
_NG = _K // _NBUF


@functools.partial(
    pl.kernel,
    mesh=plsc.VectorSubcoreMesh(core_axis_name="c", subcore_axis_name="s"),
    out_type=jax.ShapeDtypeStruct((_NC, _NS, _RPS, _D), jnp.float32),
    scratch_types=(
        [pltpu.VMEM((_NBUF, _C), jnp.int32)] * 4
        + [pltpu.VMEM((_C, _D), jnp.float32)] * _NBUF
        + [pltpu.SemaphoreType.DMA] * (2 * _NBUF + 1)
        + [pltpu.VMEM_SHARED((_N, _D), jnp.float32)]
    ),
)
def _edge_agg(h_hbm, src_hbm, dst_hbm, zeros_hbm, out_hbm, *rest):
    sidx = rest[0:2]
    didx = rest[2:4]
    bufs = rest[4:4 + _NBUF]
    gsems = rest[4 + _NBUF:4 + 2 * _NBUF]
    ssems = rest[4 + 2 * _NBUF:4 + 3 * _NBUF]
    isem = rest[4 + 3 * _NBUF]
    acc_sh = rest[5 + 3 * _NBUF]
    c = lax.axis_index("c")
    s = lax.axis_index("s")
    wid = s * _NC + c
    pltpu.sync_copy(zeros_hbm, acc_sh.at[pl.ds(s * _RPS, _RPS)])
    pltpu.sync_copy(src_hbm.at[wid, 0], sidx[0])
    pltpu.sync_copy(dst_hbm.at[wid, 0], didx[0])
    plsc.subcore_barrier()

    for b in range(_NBUF):
        pltpu.async_copy(h_hbm.at[sidx[0].at[b]], bufs[b], gsems[b])

    def _prefetch_idx(g, slot):
        pltpu.async_copy(src_hbm.at[wid, g], sidx[slot], isem)
        pltpu.async_copy(dst_hbm.at[wid, g], didx[slot], isem)

    def _wait_idx(slot):
        pltpu.make_async_copy(src_hbm.at[wid, 0], sidx[slot], isem).wait()
        pltpu.make_async_copy(dst_hbm.at[wid, 0], didx[slot], isem).wait()

    def _group(slot, fire_next, next_slot):
        for b in range(_NBUF):
            pltpu.make_async_copy(h_hbm.at[pl.ds(0, _C)], bufs[b],
                                  gsems[b]).wait()
            pltpu.async_copy(bufs[b], acc_sh.at[didx[slot].at[b]], ssems[b],
                             add=True)
            pltpu.make_async_copy(bufs[b], acc_sh.at[didx[slot].at[b]],
                                  ssems[b]).wait()
            def _refill(b=b):
                if b == 0:
                    _wait_idx(next_slot)
                pltpu.async_copy(h_hbm.at[sidx[next_slot].at[b]],
                                 bufs[b], gsems[b])

            if fire_next is True:
                _refill()
            else:
                pl.when(fire_next)(_refill)

    def outer(t, carry):
        _prefetch_idx(2 * t + 1, 1)
        _group(0, True, 1)
        not_last = t + 1 < _NG // 2

        @pl.when(not_last)
        def _():
            _prefetch_idx(2 * t + 2, 0)

        _group(1, not_last, 0)
        return carry

    lax.fori_loop(0, _NG // 2, outer, 0)
    plsc.subcore_barrier()
    pltpu.sync_copy(acc_sh.at[pl.ds(s * _RPS, _RPS)], out_hbm.at[c, s])


def _dense_body(a_ref, h_ref, wl_ref, wr_ref, bl_ref, o_ref):
    agg = a_ref[0] + a_ref[1]
    o_ref[...] = jnp.maximum(
        jax.lax.dot(agg, wl_ref[...], precision=_PREC)
        + jax.lax.dot(h_ref[...], wr_ref[...], precision=_PREC)
        + bl_ref[...],
        0.0,
    )


def _dense(a, h, wl, wr, bl):
    bs = 1000
    grid = _N // bs
    return pl.pallas_call(
        _dense_body,
        grid=(grid,),
        in_specs=[
            pl.BlockSpec((_NC, bs, _D), lambda i: (0, i, 0)),
            pl.BlockSpec((bs, _D), lambda i: (i, 0)),
            pl.BlockSpec((_D, _D), lambda i: (0, 0)),
            pl.BlockSpec((_D, _D), lambda i: (0, 0)),
            pl.BlockSpec((1, _D), lambda i: (0, 0)),
        ],
        out_specs=pl.BlockSpec((bs, _D), lambda i: (i, 0)),
        out_shape=jax.ShapeDtypeStruct((_N, _D), jnp.float32),
    )(a, h, wl, wr, bl)


def _pool_head_body(h_ref, b_ref, strat_ref, w1a_ref, w1b_ref, b1_ref,
                    w2_ref, b2_ref, w3_ref, b3_ref, o_ref, g_acc):
    i = pl.program_id(0)

    @pl.when(i == 0)
    def _():
        g_acc[...] = jnp.zeros_like(g_acc)

    bs = h_ref.shape[0]
    batch_row = b_ref[0]
    giota = jax.lax.broadcasted_iota(jnp.int32, (_B, bs), 0)
    mask = jnp.where(giota == batch_row, 1.0, 0.0)
    g_acc[...] += jax.lax.dot(mask, h_ref[...], precision=_PREC)

    @pl.when(i == pl.num_programs(0) - 1)
    def _():
        o1 = (jax.lax.dot(g_acc[...], w1a_ref[...], precision=_PREC)
              + strat_ref[...] * w1b_ref[...] + b1_ref[...])
        o1 = jnp.maximum(o1, 0.0)
        o2 = jnp.maximum(
            jax.lax.dot(o1, w2_ref[...], precision=_PREC) + b2_ref[...], 0.0)
        o_ref[...] = jax.lax.dot(o2, w3_ref[...], precision=_PREC) + b3_ref[...]


def _pool_head(h, batch3, strategy, w1a, w1b, b1, w2, b2, w3, b3):
    bs = 1000
    grid = _N // bs
    return pl.pallas_call(
        _pool_head_body,
        grid=(grid,),
        in_specs=[
            pl.BlockSpec((bs, _D), lambda i: (i, 0)),
            pl.BlockSpec((1, 1, bs), lambda i: (i, 0, 0)),
            pl.BlockSpec((_B, 1), lambda i: (0, 0)),
            pl.BlockSpec((_D, 64), lambda i: (0, 0)),
            pl.BlockSpec((1, 64), lambda i: (0, 0)),
            pl.BlockSpec((1, 64), lambda i: (0, 0)),
            pl.BlockSpec((64, 32), lambda i: (0, 0)),
            pl.BlockSpec((1, 32), lambda i: (0, 0)),
            pl.BlockSpec((32, 6), lambda i: (0, 0)),
            pl.BlockSpec((1, 6), lambda i: (0, 0)),
        ],
        out_specs=pl.BlockSpec((_B, 6), lambda i: (0, 0)),
        out_shape=jax.ShapeDtypeStruct((_B, 6), jnp.float32),
        scratch_shapes=[pltpu.VMEM((_B, _D), jnp.float32)],
    )(h, batch3, strategy, w1a, w1b, b1, w2, b2, w3, b3)


def kernel(x, edge_index, batch, strategy, Wl0, bl0, Wr0, Wl1, bl1, Wr1,
           W1, b1, W2, b2, W3, b3):
    src = edge_index[0].reshape(_NW, _NG, _NBUF, _C)
    dst = edge_index[1].reshape(_NW, _NG, _NBUF, _C)
    zeros = jnp.zeros((_RPS, _D), jnp.float32)

    a0 = _edge_agg(x, src, dst, zeros).reshape(_NC, _N, _D)
    h1 = _dense(a0, x, Wl0, Wr0, bl0.reshape(1, _D))
    a1 = _edge_agg(h1, src, dst, zeros).reshape(_NC, _N, _D)
    h2 = _dense(a1, h1, Wl1, Wr1, bl1.reshape(1, _D))

    batch3 = batch.reshape(_N // 1000, 1, 1000)
    return _pool_head(h2, batch3, strategy,
                      W1[:_D], W1[_D:], b1.reshape(1, 64),
                      W2, b2.reshape(1, 32), W3, b3.reshape(1, 6))


# speedup vs baseline: 11.7822x; 1.0899x over previous
import functools

import jax
import jax.numpy as jnp
from jax import lax
from jax.experimental import pallas as pl
from jax.experimental.pallas import tpu as pltpu
from jax.experimental.pallas import tpu_sc as plsc

_N, _E, _D, _B = 10000, 320000, 128, 64
_NC, _NS = 2, 16
_NW = _NC * _NS
_EW = _E // _NW
_C = 40
_K = _EW // _C
_NBUF = 5
_NG = _K // _NBUF

_STRIPE = 624
_LAST = _N - (_NS - 1) * _STRIPE

_PREC = jax.lax.Precision.HIGHEST


@functools.partial(
    pl.kernel,
    mesh=plsc.VectorSubcoreMesh(core_axis_name="c", subcore_axis_name="s"),
    out_type=jax.ShapeDtypeStruct((_NC, _N, _D), jnp.float32),
    scratch_types=(
        [pltpu.VMEM((_NBUF, _C), jnp.int32)] * 4
        + [pltpu.VMEM((_C, _D), jnp.float32)] * _NBUF
        + [pltpu.SemaphoreType.DMA] * (2 * _NBUF + 1)
        + [pltpu.VMEM_SHARED((_N, _D), jnp.float32)]
    ),
)
def _edge_agg(h_hbm, edge_hbm, zeros_hbm, out_hbm, *rest):
    sidx = rest[0:2]
    didx = rest[2:4]
    bufs = rest[4:4 + _NBUF]
    gsems = rest[4 + _NBUF:4 + 2 * _NBUF]
    ssems = rest[4 + 2 * _NBUF:4 + 3 * _NBUF]
    isem = rest[4 + 3 * _NBUF]
    acc_sh = rest[5 + 3 * _NBUF]
    c = lax.axis_index("c")
    s = lax.axis_index("s")
    wid = s * _NC + c

    for t in range(_NS):
        rows = _STRIPE if t < _NS - 1 else _LAST

        @pl.when(s == t)
        def _(t=t, rows=rows):
            pltpu.sync_copy(zeros_hbm.at[pl.ds(0, rows)],
                            acc_sh.at[pl.ds(t * _STRIPE, rows)])

    pltpu.sync_copy(edge_hbm.at[0, wid, 0], sidx[0])
    pltpu.sync_copy(edge_hbm.at[1, wid, 0], didx[0])
    plsc.subcore_barrier()

    for b in range(_NBUF):
        pltpu.async_copy(h_hbm.at[sidx[0].at[b]], bufs[b], gsems[b])

    def _prefetch_idx(g, slot):
        pltpu.async_copy(edge_hbm.at[0, wid, g], sidx[slot], isem)
        pltpu.async_copy(edge_hbm.at[1, wid, g], didx[slot], isem)

    def _wait_idx(slot):
        pltpu.make_async_copy(edge_hbm.at[0, wid, 0], sidx[slot], isem).wait()
        pltpu.make_async_copy(edge_hbm.at[1, wid, 0], didx[slot], isem).wait()

    def _group(slot, fire_next, next_slot):
        for b in range(_NBUF):
            pltpu.make_async_copy(h_hbm.at[pl.ds(0, _C)], bufs[b],
                                  gsems[b]).wait()
            pltpu.async_copy(bufs[b], acc_sh.at[didx[slot].at[b]], ssems[b],
                             add=True)
            pltpu.make_async_copy(bufs[b], acc_sh.at[didx[slot].at[b]],
                                  ssems[b]).wait()

            def _refill(b=b):
                if b == 0:
                    _wait_idx(next_slot)
                pltpu.async_copy(h_hbm.at[sidx[next_slot].at[b]],
                                 bufs[b], gsems[b])

            if fire_next is True:
                _refill()
            else:
                pl.when(fire_next)(_refill)

    def outer(t, carry):
        _prefetch_idx(2 * t + 1, 1)
        _group(0, True, 1)
        not_last = t + 1 < _NG // 2

        @pl.when(not_last)
        def _():
            _prefetch_idx(2 * t + 2, 0)

        _group(1, not_last, 0)
        return carry

    lax.fori_loop(0, _NG // 2, outer, 0)
    plsc.subcore_barrier()

    for t in range(_NS):
        rows = _STRIPE if t < _NS - 1 else _LAST

        @pl.when(s == t)
        def _(t=t, rows=rows):
            pltpu.sync_copy(acc_sh.at[pl.ds(t * _STRIPE, rows)],
                            out_hbm.at[c, pl.ds(t * _STRIPE, rows)])


def _dense_body(a_ref, h_ref, wl_ref, wr_ref, bl_ref, o_ref):
    agg = a_ref[0] + a_ref[1]
    o_ref[...] = jnp.maximum(
        jax.lax.dot(agg, wl_ref[...], precision=_PREC)
        + jax.lax.dot(h_ref[...], wr_ref[...], precision=_PREC)
        + bl_ref[...],
        0.0,
    )


def _dense(a, h, wl, wr, bl):
    bs = 1000
    grid = _N // bs
    return pl.pallas_call(
        _dense_body,
        grid=(grid,),
        in_specs=[
            pl.BlockSpec((_NC, bs, _D), lambda i: (0, i, 0)),
            pl.BlockSpec((bs, _D), lambda i: (i, 0)),
            pl.BlockSpec((_D, _D), lambda i: (0, 0)),
            pl.BlockSpec((_D, _D), lambda i: (0, 0)),
            pl.BlockSpec((1, _D), lambda i: (0, 0)),
        ],
        out_specs=pl.BlockSpec((bs, _D), lambda i: (i, 0)),
        out_shape=jax.ShapeDtypeStruct((_N, _D), jnp.float32),
    )(a, h, wl, wr, bl)


def _pool_head_body(h_ref, b_ref, strat_ref, w1a_ref, w1b_ref, b1_ref,
                    w2_ref, b2_ref, w3_ref, b3_ref, o_ref, g_acc):
    i = pl.program_id(0)

    @pl.when(i == 0)
    def _():
        g_acc[...] = jnp.zeros_like(g_acc)

    bs = h_ref.shape[0]
    batch_row = b_ref[0]
    giota = jax.lax.broadcasted_iota(jnp.int32, (_B, bs), 0)
    mask = jnp.where(giota == batch_row, 1.0, 0.0)
    g_acc[...] += jax.lax.dot(mask, h_ref[...], precision=_PREC)

    @pl.when(i == pl.num_programs(0) - 1)
    def _():
        o1 = (jax.lax.dot(g_acc[...], w1a_ref[...], precision=_PREC)
              + strat_ref[...] * w1b_ref[...] + b1_ref[...])
        o1 = jnp.maximum(o1, 0.0)
        o2 = jnp.maximum(
            jax.lax.dot(o1, w2_ref[...], precision=_PREC) + b2_ref[...], 0.0)
        o_ref[...] = jax.lax.dot(o2, w3_ref[...], precision=_PREC) + b3_ref[...]


def _pool_head(h, batch3, strategy, w1a, w1b, b1, w2, b2, w3, b3):
    bs = 1000
    grid = _N // bs
    return pl.pallas_call(
        _pool_head_body,
        grid=(grid,),
        in_specs=[
            pl.BlockSpec((bs, _D), lambda i: (i, 0)),
            pl.BlockSpec((1, 1, bs), lambda i: (i, 0, 0)),
            pl.BlockSpec((_B, 1), lambda i: (0, 0)),
            pl.BlockSpec((_D, 64), lambda i: (0, 0)),
            pl.BlockSpec((1, 64), lambda i: (0, 0)),
            pl.BlockSpec((1, 64), lambda i: (0, 0)),
            pl.BlockSpec((64, 32), lambda i: (0, 0)),
            pl.BlockSpec((1, 32), lambda i: (0, 0)),
            pl.BlockSpec((32, 6), lambda i: (0, 0)),
            pl.BlockSpec((1, 6), lambda i: (0, 0)),
        ],
        out_specs=pl.BlockSpec((_B, 6), lambda i: (0, 0)),
        out_shape=jax.ShapeDtypeStruct((_B, 6), jnp.float32),
        scratch_shapes=[pltpu.VMEM((_B, _D), jnp.float32)],
    )(h, batch3, strategy, w1a, w1b, b1, w2, b2, w3, b3)


def kernel(x, edge_index, batch, strategy, Wl0, bl0, Wr0, Wl1, bl1, Wr1,
           W1, b1, W2, b2, W3, b3):
    edge5 = edge_index.reshape(2, _NW, _NG, _NBUF, _C)
    zeros = jnp.zeros((_LAST, _D), jnp.float32)

    a0 = _edge_agg(x, edge5, zeros)
    h1 = _dense(a0, x, Wl0, Wr0, bl0.reshape(1, _D))
    a1 = _edge_agg(h1, edge5, zeros)
    h2 = _dense(a1, h1, Wl1, Wr1, bl1.reshape(1, _D))

    batch3 = batch.reshape(_N // 1000, 1, 1000)
    return _pool_head(h2, batch3, strategy,
                      W1[:_D], W1[_D:], b1.reshape(1, 64),
                      W2, b2.reshape(1, 32), W3, b3.reshape(1, 6))


# speedup vs baseline: 13.1621x; 1.1171x over previous
import functools

import jax
import jax.numpy as jnp
from jax import lax
from jax.experimental import pallas as pl
from jax.experimental.pallas import tpu as pltpu
from jax.experimental.pallas import tpu_sc as plsc

_N, _E, _D, _B = 10000, 320000, 128, 64
_NC, _NS = 2, 16
_NW = _NC * _NS
_EW = _E // _NW
_C = 40
_K = _EW // _C
_NBUF = 5
_NG = _K // _NBUF

_STRIPE = 624
_LAST = _N - (_NS - 1) * _STRIPE

_PREC = jax.lax.Precision.HIGHEST


@functools.partial(
    pl.kernel,
    mesh=plsc.VectorSubcoreMesh(core_axis_name="c", subcore_axis_name="s"),
    out_type=jax.ShapeDtypeStruct((_NC, _N, _D), jnp.float32),
    scratch_types=(
        [pltpu.VMEM((_NBUF, _C), jnp.int32)] * 6
        + [pltpu.VMEM((_C, _D), jnp.float32)] * _NBUF
        + [pltpu.SemaphoreType.DMA] * (2 * _NBUF + 3)
        + [pltpu.VMEM_SHARED((_N, _D), jnp.float32)]
    ),
)
def _edge_agg(h_hbm, edge_hbm, zeros_hbm, out_hbm, *rest):
    sidx = rest[0:3]
    didx = rest[3:6]
    bufs = rest[6:6 + _NBUF]
    gsems = rest[6 + _NBUF:6 + 2 * _NBUF]
    ssems = rest[6 + 2 * _NBUF:6 + 3 * _NBUF]
    isems = rest[6 + 3 * _NBUF:9 + 3 * _NBUF]
    acc_sh = rest[9 + 3 * _NBUF]
    c = lax.axis_index("c")
    s = lax.axis_index("s")
    wid = s * _NC + c

    def _prefetch(g, slot):
        pltpu.async_copy(edge_hbm.at[0, wid, g], sidx[slot], isems[slot])
        pltpu.async_copy(edge_hbm.at[1, wid, g], didx[slot], isems[slot])

    def _wait_idx(slot):
        pltpu.make_async_copy(edge_hbm.at[0, wid, 0], sidx[slot],
                              isems[slot]).wait()
        pltpu.make_async_copy(edge_hbm.at[1, wid, 0], didx[slot],
                              isems[slot]).wait()

    def _fire_g(slot, row, buf):
        pltpu.async_copy(h_hbm.at[sidx[slot].at[row]], bufs[buf], gsems[buf])

    def _wait_g(buf):
        pltpu.make_async_copy(h_hbm.at[pl.ds(0, _C)], bufs[buf],
                              gsems[buf]).wait()

    def _fire_s(slot, b):
        pltpu.async_copy(bufs[b], acc_sh.at[didx[slot].at[b]], ssems[b],
                         add=True)

    def _wait_s(slot, b):
        pltpu.make_async_copy(bufs[b], acc_sh.at[didx[slot].at[b]],
                              ssems[b]).wait()

    pltpu.sync_copy(edge_hbm.at[0, wid, 0], sidx[0])
    pltpu.sync_copy(edge_hbm.at[1, wid, 0], didx[0])
    for b in range(_NBUF - 1):
        _fire_g(0, b, b)
    _prefetch(1, 1)
    for t in range(_NS):
        rows = _STRIPE if t < _NS - 1 else _LAST

        @pl.when(s == t)
        def _(t=t, rows=rows):
            pltpu.sync_copy(zeros_hbm.at[pl.ds(0, rows)],
                            acc_sh.at[pl.ds(t * _STRIPE, rows)])
    plsc.subcore_barrier()

    def _group(slot, nslot, first=False, prefetch_g=None, prefetch_slot=None,
               prefetch_when=None, fire_next=True):
        _wait_g(0)
        _fire_s(slot, 0)
        if not first:
            _wait_s(slot, _NBUF - 1)
        if prefetch_g is not None:
            if prefetch_when is None:
                _prefetch(prefetch_g, prefetch_slot)
            else:
                pl.when(prefetch_when)(
                    lambda: _prefetch(prefetch_g, prefetch_slot))
        _fire_g(slot, _NBUF - 1, _NBUF - 1)
        for b in range(1, _NBUF):
            _wait_g(b)
            _fire_s(slot, b)
            _wait_s(slot, b - 1)

            def _refill(b=b):
                if b == 1:
                    _wait_idx(nslot)
                _fire_g(nslot, b - 1, b - 1)

            if fire_next is True:
                _refill()
            else:
                pl.when(fire_next)(_refill)

    _group(0, 1, first=True, prefetch_g=2, prefetch_slot=2)
    _group(1, 2, prefetch_g=3, prefetch_slot=0)

    def body(t, carry):
        for i, (slot, nslot) in enumerate(
                [(2, 0), (0, 1), (1, 2), (2, 0), (0, 1), (1, 2)]):
            g = 2 + 6 * t + i
            _group(slot, nslot,
                   prefetch_g=g + 2, prefetch_slot=(slot + 2) % 3,
                   prefetch_when=(None if i < 4 else t < (_NG - 2) // 6 - 1),
                   fire_next=(True if i < 5 else t < (_NG - 2) // 6 - 1))
        return carry

    lax.fori_loop(0, (_NG - 2) // 6, body, 0)
    _wait_s(1, _NBUF - 1)
    plsc.subcore_barrier()

    for t in range(_NS):
        rows = _STRIPE if t < _NS - 1 else _LAST

        @pl.when(s == t)
        def _(t=t, rows=rows):
            pltpu.sync_copy(acc_sh.at[pl.ds(t * _STRIPE, rows)],
                            out_hbm.at[c, pl.ds(t * _STRIPE, rows)])


def _root_body(h_ref, wr_ref, bl_ref, o_ref):
    o_ref[...] = (jax.lax.dot(h_ref[...], wr_ref[...], precision=_PREC)
                  + bl_ref[...])


def _root(h, wr, bl):
    bs = 1000
    return pl.pallas_call(
        _root_body,
        grid=(_N // bs,),
        in_specs=[
            pl.BlockSpec((bs, _D), lambda i: (i, 0)),
            pl.BlockSpec((_D, _D), lambda i: (0, 0)),
            pl.BlockSpec((1, _D), lambda i: (0, 0)),
        ],
        out_specs=pl.BlockSpec((bs, _D), lambda i: (i, 0)),
        out_shape=jax.ShapeDtypeStruct((_N, _D), jnp.float32),
    )(h, wr, bl)


def _finish_body(a_ref, r_ref, wl_ref, o_ref):
    agg = a_ref[0] + a_ref[1]
    o_ref[...] = jnp.maximum(
        jax.lax.dot(agg, wl_ref[...], precision=_PREC) + r_ref[...], 0.0)


def _finish(a, r, wl):
    bs = 1000
    return pl.pallas_call(
        _finish_body,
        grid=(_N // bs,),
        in_specs=[
            pl.BlockSpec((_NC, bs, _D), lambda i: (0, i, 0)),
            pl.BlockSpec((bs, _D), lambda i: (i, 0)),
            pl.BlockSpec((_D, _D), lambda i: (0, 0)),
        ],
        out_specs=pl.BlockSpec((bs, _D), lambda i: (i, 0)),
        out_shape=jax.ShapeDtypeStruct((_N, _D), jnp.float32),
    )(a, r, wl)


def _pool_head_body(a_ref, r_ref, wl_ref, b_ref, strat_ref, w1_ref, b1_ref,
                    w2_ref, b2_ref, w3_ref, b3_ref, o_ref, g_acc):
    i = pl.program_id(0)

    @pl.when(i == 0)
    def _():
        g_acc[...] = jnp.zeros_like(g_acc)

    agg = a_ref[0] + a_ref[1]
    h2 = jnp.maximum(
        jax.lax.dot(agg, wl_ref[...], precision=_PREC) + r_ref[...], 0.0)
    bs = h2.shape[0]
    batch_row = b_ref[0]
    giota = jax.lax.broadcasted_iota(jnp.int32, (_B, bs), 0)
    mask = jnp.where(giota == batch_row, 1.0, 0.0)
    g_acc[...] += jax.lax.dot(mask, h2, precision=_PREC)

    @pl.when(i == pl.num_programs(0) - 1)
    def _():
        w1a = w1_ref[pl.ds(0, _D), :]
        w1b = w1_ref[pl.ds(_D, 1), :]
        o1 = (jax.lax.dot(g_acc[...], w1a, precision=_PREC)
              + strat_ref[...] * w1b + b1_ref[...])
        o1 = jnp.maximum(o1, 0.0)
        o2 = jnp.maximum(
            jax.lax.dot(o1, w2_ref[...], precision=_PREC) + b2_ref[...], 0.0)
        o_ref[...] = jax.lax.dot(o2, w3_ref[...], precision=_PREC) + b3_ref[...]


def _pool_head(a, r, wl, batch3, strategy, w1, b1, w2, b2, w3, b3):
    bs = 1000
    return pl.pallas_call(
        _pool_head_body,
        grid=(_N // bs,),
        in_specs=[
            pl.BlockSpec((_NC, bs, _D), lambda i: (0, i, 0)),
            pl.BlockSpec((bs, _D), lambda i: (i, 0)),
            pl.BlockSpec((_D, _D), lambda i: (0, 0)),
            pl.BlockSpec((1, 1, bs), lambda i: (i, 0, 0)),
            pl.BlockSpec((_B, 1), lambda i: (0, 0)),
            pl.BlockSpec((_D + 1, 64), lambda i: (0, 0)),
            pl.BlockSpec((1, 64), lambda i: (0, 0)),
            pl.BlockSpec((64, 32), lambda i: (0, 0)),
            pl.BlockSpec((1, 32), lambda i: (0, 0)),
            pl.BlockSpec((32, 6), lambda i: (0, 0)),
            pl.BlockSpec((1, 6), lambda i: (0, 0)),
        ],
        out_specs=pl.BlockSpec((_B, 6), lambda i: (0, 0)),
        out_shape=jax.ShapeDtypeStruct((_B, 6), jnp.float32),
        scratch_shapes=[pltpu.VMEM((_B, _D), jnp.float32)],
    )(a, r, wl, batch3, strategy, w1, b1, w2, b2, w3, b3)


def kernel(x, edge_index, batch, strategy, Wl0, bl0, Wr0, Wl1, bl1, Wr1,
           W1, b1, W2, b2, W3, b3):
    edge5 = edge_index.reshape(2, _NW, _NG, _NBUF, _C)
    zeros = jnp.zeros((_LAST, _D), jnp.float32)

    a0 = _edge_agg(x, edge5, zeros)
    xr = _root(x, Wr0, bl0.reshape(1, _D))
    h1 = _finish(a0, xr, Wl0)
    a1 = _edge_agg(h1, edge5, zeros)
    h1r = _root(h1, Wr1, bl1.reshape(1, _D))
    batch3 = batch.reshape(_N // 1000, 1, 1000)
    return _pool_head(a1, h1r, Wl1, batch3, strategy,
                      W1, b1.reshape(1, 64), W2, b2.reshape(1, 32),
                      W3, b3.reshape(1, 6))


# speedup vs baseline: 13.7362x; 1.0436x over previous
import functools

import jax
import jax.numpy as jnp
from jax import lax
from jax.experimental import pallas as pl
from jax.experimental.pallas import tpu as pltpu
from jax.experimental.pallas import tpu_sc as plsc

_N, _E, _D, _B = 10000, 320000, 128, 64
_NC, _NS = 2, 16
_NW = _NC * _NS
_EW = _E // _NW
_C = 40
_K = _EW // _C
_NBUF = 5
_NG = _K // _NBUF

_STRIPE = 624
_LAST = _N - (_NS - 1) * _STRIPE

_PREC = jax.lax.Precision.DEFAULT


@functools.partial(
    pl.kernel,
    mesh=plsc.VectorSubcoreMesh(core_axis_name="c", subcore_axis_name="s"),
    out_type=jax.ShapeDtypeStruct((_NC, _N, _D), jnp.float32),
    scratch_types=(
        [pltpu.VMEM((_NBUF, _C), jnp.int32)] * 6
        + [pltpu.VMEM((_C, _D), jnp.float32)] * _NBUF
        + [pltpu.SemaphoreType.DMA] * (2 * _NBUF + 3)
        + [pltpu.VMEM_SHARED((_N, _D), jnp.float32)]
    ),
)
def _edge_agg(h_hbm, edge_hbm, zeros_hbm, out_hbm, *rest):
    sidx = rest[0:3]
    didx = rest[3:6]
    bufs = rest[6:6 + _NBUF]
    gsems = rest[6 + _NBUF:6 + 2 * _NBUF]
    ssems = rest[6 + 2 * _NBUF:6 + 3 * _NBUF]
    isems = rest[6 + 3 * _NBUF:9 + 3 * _NBUF]
    acc_sh = rest[9 + 3 * _NBUF]
    c = lax.axis_index("c")
    s = lax.axis_index("s")
    wid = s * _NC + c

    def _prefetch(g, slot):
        pltpu.async_copy(edge_hbm.at[0, wid, g], sidx[slot], isems[slot])
        pltpu.async_copy(edge_hbm.at[1, wid, g], didx[slot], isems[slot])

    def _wait_idx(slot):
        pltpu.make_async_copy(edge_hbm.at[0, wid, 0], sidx[slot],
                              isems[slot]).wait()
        pltpu.make_async_copy(edge_hbm.at[1, wid, 0], didx[slot],
                              isems[slot]).wait()

    def _fire_g(slot, row, buf):
        pltpu.async_copy(h_hbm.at[sidx[slot].at[row]], bufs[buf], gsems[buf])

    def _wait_g(buf):
        pltpu.make_async_copy(h_hbm.at[pl.ds(0, _C)], bufs[buf],
                              gsems[buf]).wait()

    def _fire_s(slot, b):
        pltpu.async_copy(bufs[b], acc_sh.at[didx[slot].at[b]], ssems[b],
                         add=True)

    def _wait_s(slot, b):
        pltpu.make_async_copy(bufs[b], acc_sh.at[didx[slot].at[b]],
                              ssems[b]).wait()

    pltpu.sync_copy(edge_hbm.at[0, wid, 0], sidx[0])
    pltpu.sync_copy(edge_hbm.at[1, wid, 0], didx[0])
    for b in range(_NBUF - 1):
        _fire_g(0, b, b)
    _prefetch(1, 1)
    for t in range(_NS):
        rows = _STRIPE if t < _NS - 1 else _LAST

        @pl.when(s == t)
        def _(t=t, rows=rows):
            pltpu.sync_copy(zeros_hbm.at[pl.ds(0, rows)],
                            acc_sh.at[pl.ds(t * _STRIPE, rows)])
    plsc.subcore_barrier()

    def _group(slot, nslot, first=False, prefetch_g=None, prefetch_slot=None,
               prefetch_when=None, fire_next=True):
        _wait_g(0)
        _fire_s(slot, 0)
        if not first:
            _wait_s(slot, _NBUF - 1)
        if prefetch_g is not None:
            if prefetch_when is None:
                _prefetch(prefetch_g, prefetch_slot)
            else:
                pl.when(prefetch_when)(
                    lambda: _prefetch(prefetch_g, prefetch_slot))
        _fire_g(slot, _NBUF - 1, _NBUF - 1)
        for b in range(1, _NBUF):
            _wait_g(b)
            _fire_s(slot, b)
            _wait_s(slot, b - 1)

            def _refill(b=b):
                if b == 1:
                    _wait_idx(nslot)
                _fire_g(nslot, b - 1, b - 1)

            if fire_next is True:
                _refill()
            else:
                pl.when(fire_next)(_refill)

    _group(0, 1, first=True, prefetch_g=2, prefetch_slot=2)
    _group(1, 2, prefetch_g=3, prefetch_slot=0)

    def body(t, carry):
        for i, (slot, nslot) in enumerate(
                [(2, 0), (0, 1), (1, 2), (2, 0), (0, 1), (1, 2)]):
            g = 2 + 6 * t + i
            _group(slot, nslot,
                   prefetch_g=g + 2, prefetch_slot=(slot + 2) % 3,
                   prefetch_when=(None if i < 4 else t < (_NG - 2) // 6 - 1),
                   fire_next=(True if i < 5 else t < (_NG - 2) // 6 - 1))
        return carry

    lax.fori_loop(0, (_NG - 2) // 6, body, 0)
    _wait_s(1, _NBUF - 1)
    plsc.subcore_barrier()

    for t in range(_NS):
        rows = _STRIPE if t < _NS - 1 else _LAST

        @pl.when(s == t)
        def _(t=t, rows=rows):
            pltpu.sync_copy(acc_sh.at[pl.ds(t * _STRIPE, rows)],
                            out_hbm.at[c, pl.ds(t * _STRIPE, rows)])


def _root_body(h_ref, wr_ref, bl_ref, o_ref):
    o_ref[...] = (jax.lax.dot(h_ref[...], wr_ref[...], precision=_PREC)
                  + bl_ref[...])


def _root(h, wr, bl):
    bs = 1000
    return pl.pallas_call(
        _root_body,
        grid=(_N // bs,),
        in_specs=[
            pl.BlockSpec((bs, _D), lambda i: (i, 0)),
            pl.BlockSpec((_D, _D), lambda i: (0, 0)),
            pl.BlockSpec((1, _D), lambda i: (0, 0)),
        ],
        out_specs=pl.BlockSpec((bs, _D), lambda i: (i, 0)),
        out_shape=jax.ShapeDtypeStruct((_N, _D), jnp.float32),
    )(h, wr, bl)


def _finish_body(a_ref, r_ref, wl_ref, o_ref):
    agg = a_ref[0] + a_ref[1]
    o_ref[...] = jnp.maximum(
        jax.lax.dot(agg, wl_ref[...], precision=_PREC) + r_ref[...], 0.0)


def _finish(a, r, wl):
    bs = 1000
    return pl.pallas_call(
        _finish_body,
        grid=(_N // bs,),
        in_specs=[
            pl.BlockSpec((_NC, bs, _D), lambda i: (0, i, 0)),
            pl.BlockSpec((bs, _D), lambda i: (i, 0)),
            pl.BlockSpec((_D, _D), lambda i: (0, 0)),
        ],
        out_specs=pl.BlockSpec((bs, _D), lambda i: (i, 0)),
        out_shape=jax.ShapeDtypeStruct((_N, _D), jnp.float32),
    )(a, r, wl)


def _pool_head_body(a_ref, r_ref, wl_ref, b_ref, strat_ref, w1_ref, b1_ref,
                    w2_ref, b2_ref, w3_ref, b3_ref, o_ref, g_acc):
    i = pl.program_id(0)

    @pl.when(i == 0)
    def _():
        g_acc[...] = jnp.zeros_like(g_acc)

    agg = a_ref[0] + a_ref[1]
    h2 = jnp.maximum(
        jax.lax.dot(agg, wl_ref[...], precision=_PREC) + r_ref[...], 0.0)
    bs = h2.shape[0]
    batch_row = b_ref[0]
    giota = jax.lax.broadcasted_iota(jnp.int32, (_B, bs), 0)
    mask = jnp.where(giota == batch_row, 1.0, 0.0)
    g_acc[...] += jax.lax.dot(mask, h2, precision=_PREC)

    @pl.when(i == pl.num_programs(0) - 1)
    def _():
        w1a = w1_ref[pl.ds(0, _D), :]
        w1b = w1_ref[pl.ds(_D, 1), :]
        o1 = (jax.lax.dot(g_acc[...], w1a, precision=_PREC)
              + strat_ref[...] * w1b + b1_ref[...])
        o1 = jnp.maximum(o1, 0.0)
        o2 = jnp.maximum(
            jax.lax.dot(o1, w2_ref[...], precision=_PREC) + b2_ref[...], 0.0)
        o_ref[...] = jax.lax.dot(o2, w3_ref[...], precision=_PREC) + b3_ref[...]


def _pool_head(a, r, wl, batch3, strategy, w1, b1, w2, b2, w3, b3):
    bs = 1000
    return pl.pallas_call(
        _pool_head_body,
        grid=(_N // bs,),
        in_specs=[
            pl.BlockSpec((_NC, bs, _D), lambda i: (0, i, 0)),
            pl.BlockSpec((bs, _D), lambda i: (i, 0)),
            pl.BlockSpec((_D, _D), lambda i: (0, 0)),
            pl.BlockSpec((1, 1, bs), lambda i: (i, 0, 0)),
            pl.BlockSpec((_B, 1), lambda i: (0, 0)),
            pl.BlockSpec((_D + 1, 64), lambda i: (0, 0)),
            pl.BlockSpec((1, 64), lambda i: (0, 0)),
            pl.BlockSpec((64, 32), lambda i: (0, 0)),
            pl.BlockSpec((1, 32), lambda i: (0, 0)),
            pl.BlockSpec((32, 6), lambda i: (0, 0)),
            pl.BlockSpec((1, 6), lambda i: (0, 0)),
        ],
        out_specs=pl.BlockSpec((_B, 6), lambda i: (0, 0)),
        out_shape=jax.ShapeDtypeStruct((_B, 6), jnp.float32),
        scratch_shapes=[pltpu.VMEM((_B, _D), jnp.float32)],
    )(a, r, wl, batch3, strategy, w1, b1, w2, b2, w3, b3)


def kernel(x, edge_index, batch, strategy, Wl0, bl0, Wr0, Wl1, bl1, Wr1,
           W1, b1, W2, b2, W3, b3):
    edge5 = edge_index.reshape(2, _NW, _NG, _NBUF, _C)
    zeros = jnp.zeros((_LAST, _D), jnp.float32)

    a0 = _edge_agg(x, edge5, zeros)
    xr = _root(x, Wr0, bl0.reshape(1, _D))
    h1 = _finish(a0, xr, Wl0)
    a1 = _edge_agg(h1, edge5, zeros)
    h1r = _root(h1, Wr1, bl1.reshape(1, _D))
    batch3 = batch.reshape(_N // 1000, 1, 1000)
    return _pool_head(a1, h1r, Wl1, batch3, strategy,
                      W1, b1.reshape(1, 64), W2, b2.reshape(1, 32),
                      W3, b3.reshape(1, 6))


# speedup vs baseline: 14.2550x; 1.0378x over previous
import functools

import jax
import jax.numpy as jnp
from jax import lax
from jax.experimental import pallas as pl
from jax.experimental.pallas import tpu as pltpu
from jax.experimental.pallas import tpu_sc as plsc

_N, _E, _D, _B = 10000, 320000, 128, 64
_NC, _NS = 2, 16
_NW = _NC * _NS
_EW = _E // _NW
_C = 40
_K = _EW // _C
_NBUF = 5
_NG = _K // _NBUF

_STRIPE = 624
_LAST = _N - (_NS - 1) * _STRIPE

_PREC = jax.lax.Precision.DEFAULT


@functools.partial(
    pl.kernel,
    mesh=plsc.VectorSubcoreMesh(core_axis_name="c", subcore_axis_name="s"),
    out_type=jax.ShapeDtypeStruct((_NC, _N, _D), jnp.float32),
    scratch_types=(
        [pltpu.VMEM((_NBUF, _C), jnp.int32)] * 6
        + [pltpu.VMEM((_C, _D), jnp.float32)] * _NBUF
        + [pltpu.VMEM((16, _D), jnp.float32)]
        + [pltpu.SemaphoreType.DMA] * (2 * _NBUF + 3)
        + [pltpu.VMEM_SHARED((_N, _D), jnp.float32)]
    ),
)
def _edge_agg(h_hbm, edge_hbm, out_hbm, *rest):
    sidx = rest[0:3]
    didx = rest[3:6]
    bufs = rest[6:6 + _NBUF]
    zbuf = rest[6 + _NBUF]
    gsems = rest[7 + _NBUF:7 + 2 * _NBUF]
    ssems = rest[7 + 2 * _NBUF:7 + 3 * _NBUF]
    isems = rest[7 + 3 * _NBUF:10 + 3 * _NBUF]
    acc_sh = rest[10 + 3 * _NBUF]
    c = lax.axis_index("c")
    s = lax.axis_index("s")
    wid = s * _NC + c

    def _prefetch(g, slot):
        pltpu.async_copy(edge_hbm.at[0, wid, g], sidx[slot], isems[slot])
        pltpu.async_copy(edge_hbm.at[1, wid, g], didx[slot], isems[slot])

    def _wait_idx(slot):
        pltpu.make_async_copy(edge_hbm.at[0, wid, 0], sidx[slot],
                              isems[slot]).wait()
        pltpu.make_async_copy(edge_hbm.at[1, wid, 0], didx[slot],
                              isems[slot]).wait()

    def _fire_g(slot, row, buf):
        pltpu.async_copy(h_hbm.at[sidx[slot].at[row]], bufs[buf], gsems[buf])

    def _wait_g(buf):
        pltpu.make_async_copy(h_hbm.at[pl.ds(0, _C)], bufs[buf],
                              gsems[buf]).wait()

    def _fire_s(slot, b):
        pltpu.async_copy(bufs[b], acc_sh.at[didx[slot].at[b]], ssems[b],
                         add=True)

    def _wait_s(slot, b):
        pltpu.make_async_copy(bufs[b], acc_sh.at[didx[slot].at[b]],
                              ssems[b]).wait()

    pltpu.sync_copy(edge_hbm.at[0, wid, 0], sidx[0])
    pltpu.sync_copy(edge_hbm.at[1, wid, 0], didx[0])
    for b in range(_NBUF - 1):
        _fire_g(0, b, b)
    _prefetch(1, 1)
    def _zfill(i, carry):
        zbuf[pl.ds(lax.rem(i, 16), 1), pl.ds(16 * lax.div(i, 16), 16)] = (
            jnp.zeros((1, 16), jnp.float32))
        return carry

    lax.fori_loop(0, 16 * (_D // 16), _zfill, 0)
    nrep = _LAST // 16

    def _zrep(r, carry):
        pltpu.sync_copy(zbuf, acc_sh.at[pl.ds(s * _STRIPE + r * 16, 16)])
        return carry

    lax.fori_loop(0, lax.cond(s == _NS - 1, lambda: nrep,
                              lambda: _STRIPE // 16), _zrep, 0)
    plsc.subcore_barrier()

    def _group(slot, nslot, first=False, prefetch_g=None, prefetch_slot=None,
               prefetch_when=None, fire_next=True):
        _wait_g(0)
        _fire_s(slot, 0)
        if not first:
            _wait_s(slot, _NBUF - 1)
        if prefetch_g is not None:
            if prefetch_when is None:
                _prefetch(prefetch_g, prefetch_slot)
            else:
                pl.when(prefetch_when)(
                    lambda: _prefetch(prefetch_g, prefetch_slot))
        _fire_g(slot, _NBUF - 1, _NBUF - 1)
        for b in range(1, _NBUF):
            _wait_g(b)
            _fire_s(slot, b)
            _wait_s(slot, b - 1)

            def _refill(b=b):
                if b == 1:
                    _wait_idx(nslot)
                _fire_g(nslot, b - 1, b - 1)

            if fire_next is True:
                _refill()
            else:
                pl.when(fire_next)(_refill)

    _group(0, 1, first=True, prefetch_g=2, prefetch_slot=2)
    _group(1, 2, prefetch_g=3, prefetch_slot=0)

    def body(t, carry):
        for i, (slot, nslot) in enumerate(
                [(2, 0), (0, 1), (1, 2), (2, 0), (0, 1), (1, 2)]):
            g = 2 + 6 * t + i
            _group(slot, nslot,
                   prefetch_g=g + 2, prefetch_slot=(slot + 2) % 3,
                   prefetch_when=(None if i < 4 else t < (_NG - 2) // 6 - 1),
                   fire_next=(True if i < 5 else t < (_NG - 2) // 6 - 1))
        return carry

    lax.fori_loop(0, (_NG - 2) // 6, body, 0)
    _wait_s(1, _NBUF - 1)
    plsc.subcore_barrier()

    for t in range(_NS):
        rows = _STRIPE if t < _NS - 1 else _LAST

        @pl.when(s == t)
        def _(t=t, rows=rows):
            pltpu.sync_copy(acc_sh.at[pl.ds(t * _STRIPE, rows)],
                            out_hbm.at[c, pl.ds(t * _STRIPE, rows)])


def _root_body(h_ref, wr_ref, bl_ref, o_ref):
    o_ref[...] = (jax.lax.dot(h_ref[...], wr_ref[...], precision=_PREC)
                  + bl_ref[...])


def _root(h, wr, bl):
    bs = 1000
    return pl.pallas_call(
        _root_body,
        grid=(_N // bs,),
        in_specs=[
            pl.BlockSpec((bs, _D), lambda i: (i, 0)),
            pl.BlockSpec((_D, _D), lambda i: (0, 0)),
            pl.BlockSpec((_D,), lambda i: (0,)),
        ],
        out_specs=pl.BlockSpec((bs, _D), lambda i: (i, 0)),
        out_shape=jax.ShapeDtypeStruct((_N, _D), jnp.float32),
    )(h, wr, bl)


def _finish_body(a_ref, r_ref, wl_ref, o_ref):
    agg = a_ref[0] + a_ref[1]
    o_ref[...] = jnp.maximum(
        jax.lax.dot(agg, wl_ref[...], precision=_PREC) + r_ref[...], 0.0)


def _finish(a, r, wl):
    bs = 1000
    return pl.pallas_call(
        _finish_body,
        grid=(_N // bs,),
        in_specs=[
            pl.BlockSpec((_NC, bs, _D), lambda i: (0, i, 0)),
            pl.BlockSpec((bs, _D), lambda i: (i, 0)),
            pl.BlockSpec((_D, _D), lambda i: (0, 0)),
        ],
        out_specs=pl.BlockSpec((bs, _D), lambda i: (i, 0)),
        out_shape=jax.ShapeDtypeStruct((_N, _D), jnp.float32),
    )(a, r, wl)


def _pool_head_body(a_ref, r_ref, wl_ref, b_ref, strat_ref, w1_ref, b1_ref,
                    w2_ref, b2_ref, w3_ref, b3_ref, o_ref, g_acc):
    i = pl.program_id(0)

    @pl.when(i == 0)
    def _():
        g_acc[...] = jnp.zeros_like(g_acc)

    agg = a_ref[0] + a_ref[1]
    h2 = jnp.maximum(
        jax.lax.dot(agg, wl_ref[...], precision=_PREC) + r_ref[...], 0.0)
    bs = h2.shape[0]
    batch_row = b_ref[0]
    giota = jax.lax.broadcasted_iota(jnp.int32, (_B, bs), 0)
    mask = jnp.where(giota == batch_row, 1.0, 0.0)
    g_acc[...] += jax.lax.dot(mask, h2, precision=_PREC)

    @pl.when(i == pl.num_programs(0) - 1)
    def _():
        w1a = w1_ref[pl.ds(0, _D), :]
        w1b = w1_ref[pl.ds(_D, 1), :]
        o1 = (jax.lax.dot(g_acc[...], w1a, precision=_PREC)
              + strat_ref[...] * w1b + b1_ref[...])
        o1 = jnp.maximum(o1, 0.0)
        o2 = jnp.maximum(
            jax.lax.dot(o1, w2_ref[...], precision=_PREC) + b2_ref[...], 0.0)
        o_ref[...] = jax.lax.dot(o2, w3_ref[...], precision=_PREC) + b3_ref[...]


def _pool_head(a, r, wl, batch3, strategy, w1, b1, w2, b2, w3, b3):
    bs = 1000
    return pl.pallas_call(
        _pool_head_body,
        grid=(_N // bs,),
        in_specs=[
            pl.BlockSpec((_NC, bs, _D), lambda i: (0, i, 0)),
            pl.BlockSpec((bs, _D), lambda i: (i, 0)),
            pl.BlockSpec((_D, _D), lambda i: (0, 0)),
            pl.BlockSpec((1, 1, bs), lambda i: (i, 0, 0)),
            pl.BlockSpec((_B, 1), lambda i: (0, 0)),
            pl.BlockSpec((_D + 1, 64), lambda i: (0, 0)),
            pl.BlockSpec((64,), lambda i: (0,)),
            pl.BlockSpec((64, 32), lambda i: (0, 0)),
            pl.BlockSpec((32,), lambda i: (0,)),
            pl.BlockSpec((32, 6), lambda i: (0, 0)),
            pl.BlockSpec((6,), lambda i: (0,)),
        ],
        out_specs=pl.BlockSpec((_B, 6), lambda i: (0, 0)),
        out_shape=jax.ShapeDtypeStruct((_B, 6), jnp.float32),
        scratch_shapes=[pltpu.VMEM((_B, _D), jnp.float32)],
    )(a, r, wl, batch3, strategy, w1, b1, w2, b2, w3, b3)


def kernel(x, edge_index, batch, strategy, Wl0, bl0, Wr0, Wl1, bl1, Wr1,
           W1, b1, W2, b2, W3, b3):
    edge5 = edge_index.reshape(2, _NW, _NG, _NBUF, _C)
    a0 = _edge_agg(x, edge5)
    xr = _root(x, Wr0, bl0)
    h1 = _finish(a0, xr, Wl0)
    a1 = _edge_agg(h1, edge5)
    h1r = _root(h1, Wr1, bl1)
    batch3 = batch.reshape(_N // 1000, 1, 1000)
    return _pool_head(a1, h1r, Wl1, batch3, strategy,
                      W1, b1, W2, b2, W3, b3)


# speedup vs baseline: 14.5413x; 1.0201x over previous
import functools

import jax
import jax.numpy as jnp
from jax import lax
from jax.experimental import pallas as pl
from jax.experimental.pallas import tpu as pltpu
from jax.experimental.pallas import tpu_sc as plsc

_N, _E, _D, _B = 10000, 320000, 128, 64
_NC, _NS = 2, 16
_NW = _NC * _NS
_EW = _E // _NW
_C = 40
_K = _EW // _C
_NBUF = 5
_NG = _K // _NBUF

_STRIPE = 624
_LAST = _N - (_NS - 1) * _STRIPE

_PREC = jax.lax.Precision.DEFAULT


@functools.partial(
    pl.kernel,
    mesh=plsc.VectorSubcoreMesh(core_axis_name="c", subcore_axis_name="s"),
    out_type=jax.ShapeDtypeStruct((_NC, _N, _D), jnp.float32),
    scratch_types=(
        [pltpu.VMEM((_NBUF, _C), jnp.int32)] * 6
        + [pltpu.VMEM((_C, _D), jnp.float32)] * _NBUF
        + [pltpu.VMEM((16, _D), jnp.float32)]
        + [pltpu.SemaphoreType.DMA] * (2 * _NBUF + 3)
        + [pltpu.VMEM_SHARED((_N, _D), jnp.float32)]
    ),
)
def _edge_agg(h_hbm, edge_hbm, out_hbm, *rest):
    sidx = rest[0:3]
    didx = rest[3:6]
    bufs = rest[6:6 + _NBUF]
    zbuf = rest[6 + _NBUF]
    gsems = rest[7 + _NBUF:7 + 2 * _NBUF]
    ssems = rest[7 + 2 * _NBUF:7 + 3 * _NBUF]
    isems = rest[7 + 3 * _NBUF:10 + 3 * _NBUF]
    acc_sh = rest[10 + 3 * _NBUF]
    c = lax.axis_index("c")
    s = lax.axis_index("s")
    wid = s * _NC + c

    def _prefetch(g, slot):
        pltpu.async_copy(edge_hbm.at[0, wid, g], sidx[slot], isems[slot])
        pltpu.async_copy(edge_hbm.at[1, wid, g], didx[slot], isems[slot])

    def _wait_idx(slot):
        pltpu.make_async_copy(edge_hbm.at[0, wid, 0], sidx[slot],
                              isems[slot]).wait()
        pltpu.make_async_copy(edge_hbm.at[1, wid, 0], didx[slot],
                              isems[slot]).wait()

    def _fire_g(slot, row, buf):
        pltpu.async_copy(h_hbm.at[sidx[slot].at[row]], bufs[buf], gsems[buf])

    def _wait_g(buf):
        pltpu.make_async_copy(h_hbm.at[pl.ds(0, _C)], bufs[buf],
                              gsems[buf]).wait()

    def _fire_s(slot, b):
        pltpu.async_copy(bufs[b], acc_sh.at[didx[slot].at[b]], ssems[b],
                         add=True)

    def _wait_s(slot, b):
        pltpu.make_async_copy(bufs[b], acc_sh.at[didx[slot].at[b]],
                              ssems[b]).wait()

    pltpu.sync_copy(edge_hbm.at[0, wid, 0], sidx[0])
    pltpu.sync_copy(edge_hbm.at[1, wid, 0], didx[0])
    for b in range(_NBUF - 1):
        _fire_g(0, b, b)
    _prefetch(1, 1)
    def _zfill(i, carry):
        zbuf[pl.ds(lax.rem(i, 16), 1), pl.ds(16 * lax.div(i, 16), 16)] = (
            jnp.zeros((1, 16), jnp.float32))
        return carry

    lax.fori_loop(0, 16 * (_D // 16), _zfill, 0)
    nrep = _LAST // 16

    def _zrep(r, carry):
        pltpu.sync_copy(zbuf, acc_sh.at[pl.ds(s * _STRIPE + r * 16, 16)])
        return carry

    lax.fori_loop(0, lax.cond(s == _NS - 1, lambda: nrep,
                              lambda: _STRIPE // 16), _zrep, 0)
    plsc.subcore_barrier()

    def _group(slot, nslot, first=False, prefetch_g=None, prefetch_slot=None,
               prefetch_when=None, fire_next=True):
        _wait_g(0)
        _fire_s(slot, 0)
        if not first:
            _wait_s(slot, _NBUF - 1)
        if prefetch_g is not None:
            if prefetch_when is None:
                _prefetch(prefetch_g, prefetch_slot)
            else:
                pl.when(prefetch_when)(
                    lambda: _prefetch(prefetch_g, prefetch_slot))
        _fire_g(slot, _NBUF - 1, _NBUF - 1)
        for b in range(1, _NBUF):
            _wait_g(b)
            _fire_s(slot, b)
            _wait_s(slot, b - 1)

            def _refill(b=b):
                if b == 1:
                    _wait_idx(nslot)
                _fire_g(nslot, b - 1, b - 1)

            if fire_next is True:
                _refill()
            else:
                pl.when(fire_next)(_refill)

    _group(0, 1, first=True, prefetch_g=2, prefetch_slot=2)
    _group(1, 2, prefetch_g=3, prefetch_slot=0)

    def body(t, carry):
        for i, (slot, nslot) in enumerate(
                [(2, 0), (0, 1), (1, 2), (2, 0), (0, 1), (1, 2)]):
            g = 2 + 6 * t + i
            _group(slot, nslot,
                   prefetch_g=g + 2, prefetch_slot=(slot + 2) % 3,
                   prefetch_when=(None if i < 4 else t < (_NG - 2) // 6 - 1),
                   fire_next=(True if i < 5 else t < (_NG - 2) // 6 - 1))
        return carry

    lax.fori_loop(0, (_NG - 2) // 6, body, 0)
    _wait_s(1, _NBUF - 1)
    plsc.subcore_barrier()

    for t in range(_NS):
        rows = _STRIPE if t < _NS - 1 else _LAST

        @pl.when(s == t)
        def _(t=t, rows=rows):
            pltpu.sync_copy(acc_sh.at[pl.ds(t * _STRIPE, rows)],
                            out_hbm.at[c, pl.ds(t * _STRIPE, rows)])


def _root_body(h_ref, wr_ref, bl_ref, o_ref):
    o_ref[...] = (jax.lax.dot(h_ref[...], wr_ref[...], precision=_PREC)
                  + bl_ref[...])


def _root(h, wr, bl):
    bs = 2000
    return pl.pallas_call(
        _root_body,
        grid=(_N // bs,),
        in_specs=[
            pl.BlockSpec((bs, _D), lambda i: (i, 0)),
            pl.BlockSpec((_D, _D), lambda i: (0, 0)),
            pl.BlockSpec((_D,), lambda i: (0,)),
        ],
        out_specs=pl.BlockSpec((bs, _D), lambda i: (i, 0)),
        out_shape=jax.ShapeDtypeStruct((_N, _D), jnp.float32),
    )(h, wr, bl)


def _finish_body(a_ref, r_ref, wl_ref, o_ref):
    agg = a_ref[0] + a_ref[1]
    o_ref[...] = jnp.maximum(
        jax.lax.dot(agg, wl_ref[...], precision=_PREC) + r_ref[...], 0.0)


def _finish(a, r, wl):
    bs = 2000
    return pl.pallas_call(
        _finish_body,
        grid=(_N // bs,),
        in_specs=[
            pl.BlockSpec((_NC, bs, _D), lambda i: (0, i, 0)),
            pl.BlockSpec((bs, _D), lambda i: (i, 0)),
            pl.BlockSpec((_D, _D), lambda i: (0, 0)),
        ],
        out_specs=pl.BlockSpec((bs, _D), lambda i: (i, 0)),
        out_shape=jax.ShapeDtypeStruct((_N, _D), jnp.float32),
    )(a, r, wl)


def _pool_head_body(a_ref, r_ref, wl_ref, b_ref, strat_ref, w1_ref, b1_ref,
                    w2_ref, b2_ref, w3_ref, b3_ref, o_ref, g_acc):
    i = pl.program_id(0)

    @pl.when(i == 0)
    def _():
        g_acc[...] = jnp.zeros_like(g_acc)

    agg = a_ref[0] + a_ref[1]
    h2 = jnp.maximum(
        jax.lax.dot(agg, wl_ref[...], precision=_PREC) + r_ref[...], 0.0)
    bs = h2.shape[0]
    batch_row = b_ref[0]
    giota = jax.lax.broadcasted_iota(jnp.int32, (_B, bs), 0)
    mask = jnp.where(giota == batch_row, 1.0, 0.0)
    g_acc[...] += jax.lax.dot(mask, h2, precision=_PREC)

    @pl.when(i == pl.num_programs(0) - 1)
    def _():
        w1a = w1_ref[pl.ds(0, _D), :]
        w1b = w1_ref[pl.ds(_D, 1), :]
        o1 = (jax.lax.dot(g_acc[...], w1a, precision=_PREC)
              + strat_ref[...] * w1b + b1_ref[...])
        o1 = jnp.maximum(o1, 0.0)
        o2 = jnp.maximum(
            jax.lax.dot(o1, w2_ref[...], precision=_PREC) + b2_ref[...], 0.0)
        o_ref[...] = jax.lax.dot(o2, w3_ref[...], precision=_PREC) + b3_ref[...]


def _pool_head(a, r, wl, batch3, strategy, w1, b1, w2, b2, w3, b3):
    bs = 2000
    return pl.pallas_call(
        _pool_head_body,
        grid=(_N // bs,),
        in_specs=[
            pl.BlockSpec((_NC, bs, _D), lambda i: (0, i, 0)),
            pl.BlockSpec((bs, _D), lambda i: (i, 0)),
            pl.BlockSpec((_D, _D), lambda i: (0, 0)),
            pl.BlockSpec((1, 1, bs), lambda i: (i, 0, 0)),
            pl.BlockSpec((_B, 1), lambda i: (0, 0)),
            pl.BlockSpec((_D + 1, 64), lambda i: (0, 0)),
            pl.BlockSpec((64,), lambda i: (0,)),
            pl.BlockSpec((64, 32), lambda i: (0, 0)),
            pl.BlockSpec((32,), lambda i: (0,)),
            pl.BlockSpec((32, 6), lambda i: (0, 0)),
            pl.BlockSpec((6,), lambda i: (0,)),
        ],
        out_specs=pl.BlockSpec((_B, 6), lambda i: (0, 0)),
        out_shape=jax.ShapeDtypeStruct((_B, 6), jnp.float32),
        scratch_shapes=[pltpu.VMEM((_B, _D), jnp.float32)],
    )(a, r, wl, batch3, strategy, w1, b1, w2, b2, w3, b3)


def kernel(x, edge_index, batch, strategy, Wl0, bl0, Wr0, Wl1, bl1, Wr1,
           W1, b1, W2, b2, W3, b3):
    edge5 = edge_index.reshape(2, _NW, _NG, _NBUF, _C)
    a0 = _edge_agg(x, edge5)
    xr = _root(x, Wr0, bl0)
    h1 = _finish(a0, xr, Wl0)
    a1 = _edge_agg(h1, edge5)
    h1r = _root(h1, Wr1, bl1)
    batch3 = batch.reshape(_N // 2000, 1, 2000)
    return _pool_head(a1, h1r, Wl1, batch3, strategy,
                      W1, b1, W2, b2, W3, b3)


# speedup vs baseline: 14.7224x; 1.0125x over previous
import functools

import jax
import jax.numpy as jnp
from jax import lax
from jax.experimental import pallas as pl
from jax.experimental.pallas import tpu as pltpu
from jax.experimental.pallas import tpu_sc as plsc

_N, _E, _D, _B = 10000, 320000, 128, 64
_NC, _NS = 2, 16
_NW = _NC * _NS
_EW = _E // _NW
_C = 40
_K = _EW // _C
_NBUF = 5
_NG = _K // _NBUF

_STRIPE = 624
_LAST = _N - (_NS - 1) * _STRIPE

_PREC = jax.lax.Precision.DEFAULT


@functools.partial(
    pl.kernel,
    mesh=plsc.VectorSubcoreMesh(core_axis_name="c", subcore_axis_name="s"),
    out_type=jax.ShapeDtypeStruct((_NC, _N, _D), jnp.float32),
    scratch_types=(
        [pltpu.VMEM((_NBUF, _C), jnp.int32)] * 6
        + [pltpu.VMEM((_C, _D), jnp.float32)] * _NBUF
        + [pltpu.VMEM((16, _D), jnp.float32)]
        + [pltpu.SemaphoreType.DMA] * (2 * _NBUF + 3)
        + [pltpu.VMEM_SHARED((_N, _D), jnp.float32)]
    ),
)
def _edge_agg(h_hbm, edge_hbm, out_hbm, *rest):
    sidx = rest[0:3]
    didx = rest[3:6]
    bufs = rest[6:6 + _NBUF]
    zbuf = rest[6 + _NBUF]
    gsems = rest[7 + _NBUF:7 + 2 * _NBUF]
    ssems = rest[7 + 2 * _NBUF:7 + 3 * _NBUF]
    isems = rest[7 + 3 * _NBUF:10 + 3 * _NBUF]
    acc_sh = rest[10 + 3 * _NBUF]
    c = lax.axis_index("c")
    s = lax.axis_index("s")
    wid = s * _NC + c

    def _prefetch(g, slot):
        pltpu.async_copy(edge_hbm.at[0, wid, g], sidx[slot], isems[slot])
        pltpu.async_copy(edge_hbm.at[1, wid, g], didx[slot], isems[slot])

    def _wait_idx(slot):
        pltpu.make_async_copy(edge_hbm.at[0, wid, 0], sidx[slot],
                              isems[slot]).wait()
        pltpu.make_async_copy(edge_hbm.at[1, wid, 0], didx[slot],
                              isems[slot]).wait()

    def _fire_g(slot, row, buf):
        pltpu.async_copy(h_hbm.at[sidx[slot].at[row]], bufs[buf], gsems[buf])

    def _wait_g(buf):
        pltpu.make_async_copy(h_hbm.at[pl.ds(0, _C)], bufs[buf],
                              gsems[buf]).wait()

    def _fire_s(slot, b):
        pltpu.async_copy(bufs[b], acc_sh.at[didx[slot].at[b]], ssems[b],
                         add=True)

    def _wait_s(slot, b):
        pltpu.make_async_copy(bufs[b], acc_sh.at[didx[slot].at[b]],
                              ssems[b]).wait()

    pltpu.sync_copy(edge_hbm.at[0, wid, 0], sidx[0])
    pltpu.sync_copy(edge_hbm.at[1, wid, 0], didx[0])
    for b in range(_NBUF - 1):
        _fire_g(0, b, b)
    _prefetch(1, 1)
    def _zfill(i, carry):
        zbuf[pl.ds(lax.rem(i, 16), 1), pl.ds(16 * lax.div(i, 16), 16)] = (
            jnp.zeros((1, 16), jnp.float32))
        return carry

    lax.fori_loop(0, 16 * (_D // 16), _zfill, 0)
    nrep = _LAST // 16

    def _zrep(r, carry):
        pltpu.sync_copy(zbuf, acc_sh.at[pl.ds(s * _STRIPE + r * 16, 16)])
        return carry

    lax.fori_loop(0, lax.cond(s == _NS - 1, lambda: nrep,
                              lambda: _STRIPE // 16), _zrep, 0)
    plsc.subcore_barrier()

    def _group(slot, nslot, first=False, prefetch_g=None, prefetch_slot=None,
               prefetch_when=None, fire_next=True):
        _wait_g(0)
        _fire_s(slot, 0)
        if not first:
            _wait_s(slot, _NBUF - 1)
        if prefetch_g is not None:
            if prefetch_when is None:
                _prefetch(prefetch_g, prefetch_slot)
            else:
                pl.when(prefetch_when)(
                    lambda: _prefetch(prefetch_g, prefetch_slot))
        _fire_g(slot, _NBUF - 1, _NBUF - 1)
        for b in range(1, _NBUF):
            _wait_g(b)
            _fire_s(slot, b)
            _wait_s(slot, b - 1)

            def _refill(b=b):
                if b == 1:
                    _wait_idx(nslot)
                _fire_g(nslot, b - 1, b - 1)

            if fire_next is True:
                _refill()
            else:
                pl.when(fire_next)(_refill)

    _group(0, 1, first=True, prefetch_g=2, prefetch_slot=2)
    _group(1, 2, prefetch_g=3, prefetch_slot=0)

    def body(t, carry):
        for i, (slot, nslot) in enumerate(
                [(2, 0), (0, 1), (1, 2), (2, 0), (0, 1), (1, 2)]):
            g = 2 + 6 * t + i
            _group(slot, nslot,
                   prefetch_g=g + 2, prefetch_slot=(slot + 2) % 3,
                   prefetch_when=(None if i < 4 else t < (_NG - 2) // 6 - 1),
                   fire_next=(True if i < 5 else t < (_NG - 2) // 6 - 1))
        return carry

    lax.fori_loop(0, (_NG - 2) // 6, body, 0)
    _wait_s(1, _NBUF - 1)
    plsc.subcore_barrier()

    for t in range(_NS):
        rows = _STRIPE if t < _NS - 1 else _LAST

        @pl.when(s == t)
        def _(t=t, rows=rows):
            pltpu.sync_copy(acc_sh.at[pl.ds(t * _STRIPE, rows)],
                            out_hbm.at[c, pl.ds(t * _STRIPE, rows)])


def _root_body(h_ref, wr_ref, bl_ref, o_ref):
    o_ref[...] = (jax.lax.dot(h_ref[...], wr_ref[...], precision=_PREC)
                  + bl_ref[...])


def _root(h, wr, bl):
    bs = 5000
    return pl.pallas_call(
        _root_body,
        grid=(_N // bs,),
        in_specs=[
            pl.BlockSpec((bs, _D), lambda i: (i, 0)),
            pl.BlockSpec((_D, _D), lambda i: (0, 0)),
            pl.BlockSpec((_D,), lambda i: (0,)),
        ],
        out_specs=pl.BlockSpec((bs, _D), lambda i: (i, 0)),
        out_shape=jax.ShapeDtypeStruct((_N, _D), jnp.float32),
    )(h, wr, bl)


def _finish_body(a_ref, r_ref, wl_ref, o_ref):
    agg = a_ref[0] + a_ref[1]
    o_ref[...] = jnp.maximum(
        jax.lax.dot(agg, wl_ref[...], precision=_PREC) + r_ref[...], 0.0)


def _finish(a, r, wl):
    bs = 5000
    return pl.pallas_call(
        _finish_body,
        grid=(_N // bs,),
        in_specs=[
            pl.BlockSpec((_NC, bs, _D), lambda i: (0, i, 0)),
            pl.BlockSpec((bs, _D), lambda i: (i, 0)),
            pl.BlockSpec((_D, _D), lambda i: (0, 0)),
        ],
        out_specs=pl.BlockSpec((bs, _D), lambda i: (i, 0)),
        out_shape=jax.ShapeDtypeStruct((_N, _D), jnp.float32),
    )(a, r, wl)


def _pool_head_body(a_ref, r_ref, wl_ref, b_ref, strat_ref, w1_ref, b1_ref,
                    w2_ref, b2_ref, w3_ref, b3_ref, o_ref, g_acc):
    i = pl.program_id(0)

    @pl.when(i == 0)
    def _():
        g_acc[...] = jnp.zeros_like(g_acc)

    agg = a_ref[0] + a_ref[1]
    h2 = jnp.maximum(
        jax.lax.dot(agg, wl_ref[...], precision=_PREC) + r_ref[...], 0.0)
    bs = h2.shape[0]
    batch_row = b_ref[0]
    giota = jax.lax.broadcasted_iota(jnp.int32, (_B, bs), 0)
    mask = jnp.where(giota == batch_row, 1.0, 0.0)
    g_acc[...] += jax.lax.dot(mask, h2, precision=_PREC)

    @pl.when(i == pl.num_programs(0) - 1)
    def _():
        w1a = w1_ref[pl.ds(0, _D), :]
        w1b = w1_ref[pl.ds(_D, 1), :]
        o1 = (jax.lax.dot(g_acc[...], w1a, precision=_PREC)
              + strat_ref[...] * w1b + b1_ref[...])
        o1 = jnp.maximum(o1, 0.0)
        o2 = jnp.maximum(
            jax.lax.dot(o1, w2_ref[...], precision=_PREC) + b2_ref[...], 0.0)
        o_ref[...] = jax.lax.dot(o2, w3_ref[...], precision=_PREC) + b3_ref[...]


def _pool_head(a, r, wl, batch3, strategy, w1, b1, w2, b2, w3, b3):
    bs = 5000
    return pl.pallas_call(
        _pool_head_body,
        grid=(_N // bs,),
        in_specs=[
            pl.BlockSpec((_NC, bs, _D), lambda i: (0, i, 0)),
            pl.BlockSpec((bs, _D), lambda i: (i, 0)),
            pl.BlockSpec((_D, _D), lambda i: (0, 0)),
            pl.BlockSpec((1, 1, bs), lambda i: (i, 0, 0)),
            pl.BlockSpec((_B, 1), lambda i: (0, 0)),
            pl.BlockSpec((_D + 1, 64), lambda i: (0, 0)),
            pl.BlockSpec((64,), lambda i: (0,)),
            pl.BlockSpec((64, 32), lambda i: (0, 0)),
            pl.BlockSpec((32,), lambda i: (0,)),
            pl.BlockSpec((32, 6), lambda i: (0, 0)),
            pl.BlockSpec((6,), lambda i: (0,)),
        ],
        out_specs=pl.BlockSpec((_B, 6), lambda i: (0, 0)),
        out_shape=jax.ShapeDtypeStruct((_B, 6), jnp.float32),
        scratch_shapes=[pltpu.VMEM((_B, _D), jnp.float32)],
    )(a, r, wl, batch3, strategy, w1, b1, w2, b2, w3, b3)


def kernel(x, edge_index, batch, strategy, Wl0, bl0, Wr0, Wl1, bl1, Wr1,
           W1, b1, W2, b2, W3, b3):
    edge5 = edge_index.reshape(2, _NW, _NG, _NBUF, _C)
    a0 = _edge_agg(x, edge5)
    xr = _root(x, Wr0, bl0)
    h1 = _finish(a0, xr, Wl0)
    a1 = _edge_agg(h1, edge5)
    h1r = _root(h1, Wr1, bl1)
    batch3 = batch.reshape(_N // 5000, 1, 5000)
    return _pool_head(a1, h1r, Wl1, batch3, strategy,
                      W1, b1, W2, b2, W3, b3)
